# baseline (speedup 1.0000x reference)
import functools

import jax
import jax.numpy as jnp
from jax import lax
from jax.experimental import pallas as pl
from jax.experimental.pallas import tpu as pltpu
from jax.experimental.pallas import tpu_sc as plsc

HIDDEN = 64
CTX = 128
GROUP = 4
NUM_WORKERS = 32


def _sc_dots_kernel(ex_per_w, u_labels_hbm, ctx_hbm, in_emb_hbm, out_emb_hbm,
                    out_hbm, u_idx, u_rows, lbuf, ctx_rows, dots, gsem, lsem,
                    osem):
    wid = lax.axis_index("s") * 2 + lax.axis_index("c")
    base = wid * ex_per_w
    num_groups = ex_per_w // GROUP

    n_chunks = ex_per_w // 128
    for j in range(n_chunks):
        pltpu.sync_copy(u_labels_hbm.at[pl.ds(base + j * 128, 128)],
                        u_idx.at[j])
    cps = [pltpu.async_copy(in_emb_hbm.at[u_idx.at[j]],
                            u_rows.at[pl.ds(j * 128, 128)], gsem)
           for j in range(n_chunks)]
    for cp in cps:
        cp.wait()

    def issue_gathers(g, buf):
        for e in range(GROUP):
            pltpu.async_copy(out_emb_hbm.at[lbuf.at[buf, e]],
                             ctx_rows.at[buf, e], gsem)

    def drain_gathers(buf):
        for e in range(GROUP):
            pltpu.make_async_copy(out_emb_hbm.at[lbuf.at[buf, e]],
                                  ctx_rows.at[buf, e], gsem).wait()

    def unpack4(v64):
        b_lo, b_hi = plsc.unpack(v64, format=plsc.PackFormat.INTERLEAVED,
                                 preferred_element_type=jnp.bfloat16)
        a0, a1 = plsc.unpack(b_lo, format=plsc.PackFormat.INTERLEAVED)
        a2, a3 = plsc.unpack(b_hi, format=plsc.PackFormat.INTERLEAVED)
        return a0, a1, a2, a3

    def compute(g, buf):
        lane = lax.iota(jnp.int32, 16)
        for e in range(GROUP):
            b_local = g * GROUP + e
            u_vecs = unpack4(u_rows[b_local, pl.ds(0, 64)])

            def blk_body(t, _):
                acc = jnp.zeros((16,), jnp.float32)
                for r in range(16):
                    row = t * 16 + r
                    c_vecs = unpack4(ctx_rows[buf, e, row, pl.ds(0, 64)])
                    p = c_vecs[0] * u_vecs[0]
                    for h in range(1, 4):
                        p = p + c_vecs[h] * u_vecs[h]
                    acc = jnp.where(lane == r, jnp.sum(p), acc)
                dots[buf, e, pl.ds(t * 16, 16)] = acc
                return 0

            lax.fori_loop(0, CTX // 16, blk_body, 0)

    def issue_out(g, buf):
        pltpu.async_copy(dots.at[buf],
                         out_hbm.at[pl.ds(base + g * GROUP, GROUP)],
                         osem)

    def drain_out(g, buf):
        pltpu.make_async_copy(dots.at[buf],
                              out_hbm.at[pl.ds(base + g * GROUP, GROUP)],
                              osem).wait()

    def step(g, buf, has_next, has_prev_out):
        if has_next:
            lcp = pltpu.async_copy(ctx_hbm.at[pl.ds(base + (g + 1) * GROUP,
                                                    GROUP)],
                                   lbuf.at[1 - buf], lsem)
        drain_gathers(buf)
        if has_next:
            lcp.wait()
            issue_gathers(g + 1, 1 - buf)
        if has_prev_out:
            drain_out(g - 2, buf)
        compute(g, buf)
        issue_out(g, buf)

    pltpu.sync_copy(ctx_hbm.at[pl.ds(base, GROUP)], lbuf.at[0])
    issue_gathers(0, 0)
    step(0, 0, True, False)
    step(1, 1, True, False)

    def pair_body(i, _):
        step(2 * i, 0, True, True)
        step(2 * i + 1, 1, True, True)
        return 0

    lax.fori_loop(1, num_groups // 2 - 1, pair_body, 0)
    step(num_groups - 2, 0, True, True)
    step(num_groups - 1, 1, False, True)
    drain_out(num_groups - 2, 0)
    drain_out(num_groups - 1, 1)


def _sc_dots(u_labels, ctx_labels, in_emb, out_emb):
    b = u_labels.shape[0]
    ex_per_w = b // NUM_WORKERS
    mesh = plsc.VectorSubcoreMesh(core_axis_name="c", subcore_axis_name="s")
    f = pl.kernel(
        functools.partial(_sc_dots_kernel, ex_per_w),
        out_type=jax.ShapeDtypeStruct((b, CTX), jnp.float32),
        mesh=mesh,
        scratch_types=[
            pltpu.VMEM((ex_per_w // 128, 128), jnp.int32),
            pltpu.VMEM((ex_per_w, HIDDEN), jnp.float8_e4m3fn),
            pltpu.VMEM((2, GROUP, CTX), jnp.int32),
            pltpu.VMEM((2, GROUP, CTX, HIDDEN), jnp.float8_e4m3fn),
            pltpu.VMEM((2, GROUP, CTX), jnp.float32),
            pltpu.SemaphoreType.DMA,
            pltpu.SemaphoreType.DMA,
            pltpu.SemaphoreType.DMA,
        ],
        compiler_params=pltpu.CompilerParams(needs_layout_passes=False,
                                             use_tc_tiling_on_sc=False),
    )
    return f(u_labels, ctx_labels, in_emb, out_emb)


def _cast_body(a_ref, b_ref, o_ref):
    o_ref[...] = jnp.concatenate([a_ref[...], b_ref[...]], axis=1)


def _cast_compact(t):
    t8 = t.astype(jnp.float8_e4m3fn)
    v = t.shape[0]
    rows = 4096
    nblk = v // 2 // rows
    return pl.pallas_call(
        _cast_body,
        grid=(nblk,),
        in_specs=[pl.BlockSpec((rows, HIDDEN), lambda i: (i, 0)),
                  pl.BlockSpec((rows, HIDDEN), lambda i, nb=nblk: (i + nb, 0))],
        out_specs=pl.BlockSpec((rows, 2 * HIDDEN), lambda i: (i, 0)),
        out_shape=jax.ShapeDtypeStruct((v // 2, 2 * HIDDEN),
                                       jnp.float8_e4m3fn),
    )(t8, t8)


def _tc_loss_kernel(p, n, d_ref, o_ref):
    d = d_ref[...]
    col = lax.broadcasted_iota(jnp.int32, d.shape, 1)
    x = jnp.where(col < p, d, -d)
    ls = jnp.minimum(x, 0.0) - jnp.log1p(jnp.exp(-jnp.abs(x)))
    ls = jnp.where(col < p + n, ls, 0.0)
    loss = -jnp.sum(ls, axis=1)
    o_ref[...] = loss.reshape(o_ref.shape)


def _tc_loss(dots, p, n):
    b = dots.shape[0]
    blk = 2048
    out = pl.pallas_call(
        functools.partial(_tc_loss_kernel, p, n),
        grid=(b // blk,),
        in_specs=[pl.BlockSpec((blk, CTX), lambda i: (i, 0))],
        out_specs=pl.BlockSpec((blk // 128, 128), lambda i: (i, 0)),
        out_shape=jax.ShapeDtypeStruct((b // 128, 128), jnp.float32),
    )(dots)
    return out.reshape(b)


def kernel(input_labels, pos_labels, neg_labels, in_emb, out_emb):
    b, p = pos_labels.shape
    n = neg_labels.shape[1]
    pad = CTX - p - n
    ctx = jnp.concatenate(
        [pos_labels.astype(jnp.int32), neg_labels.astype(jnp.int32),
         jnp.zeros((b, pad), jnp.int32)], axis=1)
    v = in_emb.shape[0]
    half = v // 2
    in8 = _cast_compact(in_emb).reshape(v, HIDDEN)
    out8 = _cast_compact(out_emb).reshape(v, HIDDEN)

    def remap(l):
        return 2 * (l % half) + (l >= half).astype(jnp.int32)

    dots = _sc_dots(remap(input_labels.astype(jnp.int32)), remap(ctx),
                    in8, out8)
    return _tc_loss(dots, p, n)


# speedup vs baseline: 1.3083x; 1.3083x over previous
import functools

import jax
import jax.numpy as jnp
from jax import lax
from jax.experimental import pallas as pl
from jax.experimental.pallas import tpu as pltpu
from jax.experimental.pallas import tpu_sc as plsc

HIDDEN = 64
CTX = 128
GROUP = 4
NUM_WORKERS = 32


def _sc_dots_kernel(ex_per_w, u_labels_hbm, ctx_hbm, in_emb_hbm, out_emb_hbm,
                    out_hbm, u_idx, u_rows, lbuf, ctx_rows, dots, gsem, lsem,
                    osem):
    wid = lax.axis_index("s") * 2 + lax.axis_index("c")
    base = wid * ex_per_w
    num_groups = ex_per_w // GROUP

    n_chunks = ex_per_w // 128
    for j in range(n_chunks):
        pltpu.sync_copy(u_labels_hbm.at[pl.ds(base + j * 128, 128)],
                        u_idx.at[j])
    cps = [pltpu.async_copy(in_emb_hbm.at[u_idx.at[j]],
                            u_rows.at[pl.ds(j * 128, 128)], gsem)
           for j in range(n_chunks)]
    for cp in cps:
        cp.wait()

    def issue_gathers(g, buf):
        for e in range(GROUP):
            pltpu.async_copy(out_emb_hbm.at[lbuf.at[buf, e]],
                             ctx_rows.at[buf, e], gsem)

    def drain_gathers(buf):
        for e in range(GROUP):
            pltpu.make_async_copy(out_emb_hbm.at[lbuf.at[buf, e]],
                                  ctx_rows.at[buf, e], gsem).wait()

    def unpack4(v64):
        b_lo, b_hi = plsc.unpack(v64, format=plsc.PackFormat.INTERLEAVED,
                                 preferred_element_type=jnp.bfloat16)
        a0, a1 = plsc.unpack(b_lo, format=plsc.PackFormat.INTERLEAVED)
        a2, a3 = plsc.unpack(b_hi, format=plsc.PackFormat.INTERLEAVED)
        return a0, a1, a2, a3

    def compute(g, buf):
        lane = lax.iota(jnp.int32, 16)
        for e in range(GROUP):
            b_local = g * GROUP + e
            u_vecs = unpack4(u_rows[b_local, pl.ds(0, 64)])

            def blk_body(t, _):
                acc = jnp.zeros((16,), jnp.float32)
                for r in range(16):
                    row = t * 16 + r
                    c_vecs = unpack4(ctx_rows[buf, e, row, pl.ds(0, 64)])
                    p = c_vecs[0] * u_vecs[0]
                    for h in range(1, 4):
                        p = p + c_vecs[h] * u_vecs[h]
                    acc = jnp.where(lane == r, jnp.sum(p), acc)
                dots[buf, e, pl.ds(t * 16, 16)] = acc
                return 0

            lax.fori_loop(0, CTX // 16, blk_body, 0)

    def issue_out(g, buf):
        pltpu.async_copy(dots.at[buf],
                         out_hbm.at[pl.ds(base + g * GROUP, GROUP)],
                         osem)

    def drain_out(g, buf):
        pltpu.make_async_copy(dots.at[buf],
                              out_hbm.at[pl.ds(base + g * GROUP, GROUP)],
                              osem).wait()

    def step(g, buf, has_next, has_prev_out):
        if has_next:
            lcp = pltpu.async_copy(ctx_hbm.at[pl.ds(base + (g + 1) * GROUP,
                                                    GROUP)],
                                   lbuf.at[1 - buf], lsem)
        drain_gathers(buf)
        if has_next:
            lcp.wait()
            issue_gathers(g + 1, 1 - buf)
        if has_prev_out:
            drain_out(g - 2, buf)
        compute(g, buf)
        issue_out(g, buf)

    pltpu.sync_copy(ctx_hbm.at[pl.ds(base, GROUP)], lbuf.at[0])
    issue_gathers(0, 0)
    step(0, 0, True, False)
    step(1, 1, True, False)

    def pair_body(i, _):
        step(2 * i, 0, True, True)
        step(2 * i + 1, 1, True, True)
        return 0

    lax.fori_loop(1, num_groups // 2 - 1, pair_body, 0)
    step(num_groups - 2, 0, True, True)
    step(num_groups - 1, 1, False, True)
    drain_out(num_groups - 2, 0)
    drain_out(num_groups - 1, 1)


def _sc_dots(u_labels, ctx_labels, in_emb, out_emb):
    b = u_labels.shape[0]
    ex_per_w = b // NUM_WORKERS
    mesh = plsc.VectorSubcoreMesh(core_axis_name="c", subcore_axis_name="s")
    f = pl.kernel(
        functools.partial(_sc_dots_kernel, ex_per_w),
        out_type=jax.ShapeDtypeStruct((b, CTX), jnp.float32),
        mesh=mesh,
        scratch_types=[
            pltpu.VMEM((ex_per_w // 128, 128), jnp.int32),
            pltpu.VMEM((ex_per_w, HIDDEN), jnp.float8_e4m3fn),
            pltpu.VMEM((2, GROUP, CTX), jnp.int32),
            pltpu.VMEM((2, GROUP, CTX, HIDDEN), jnp.float8_e4m3fn),
            pltpu.VMEM((2, GROUP, CTX), jnp.float32),
            pltpu.SemaphoreType.DMA,
            pltpu.SemaphoreType.DMA,
            pltpu.SemaphoreType.DMA,
        ],
        compiler_params=pltpu.CompilerParams(needs_layout_passes=False,
                                             use_tc_tiling_on_sc=False),
    )
    return f(u_labels, ctx_labels, in_emb, out_emb)


def _tc_loss_kernel(p, n, d_ref, o_ref):
    d = d_ref[...]
    col = lax.broadcasted_iota(jnp.int32, d.shape, 1)
    x = jnp.where(col < p, d, -d)
    ls = jnp.minimum(x, 0.0) - jnp.log1p(jnp.exp(-jnp.abs(x)))
    ls = jnp.where(col < p + n, ls, 0.0)
    loss = -jnp.sum(ls, axis=1)
    o_ref[...] = loss.reshape(o_ref.shape)


def _tc_loss(dots, p, n):
    b = dots.shape[0]
    blk = 2048
    out = pl.pallas_call(
        functools.partial(_tc_loss_kernel, p, n),
        grid=(b // blk,),
        in_specs=[pl.BlockSpec((blk, CTX), lambda i: (i, 0))],
        out_specs=pl.BlockSpec((blk // 128, 128), lambda i: (i, 0)),
        out_shape=jax.ShapeDtypeStruct((b // 128, 128), jnp.float32),
    )(dots)
    return out.reshape(b)


def kernel(input_labels, pos_labels, neg_labels, in_emb, out_emb):
    b, p = pos_labels.shape
    n = neg_labels.shape[1]
    pad = CTX - p - n
    ctx = jnp.concatenate(
        [pos_labels.astype(jnp.int32), neg_labels.astype(jnp.int32),
         jnp.zeros((b, pad), jnp.int32)], axis=1)
    dots = _sc_dots(input_labels.astype(jnp.int32), ctx,
                    in_emb.astype(jnp.float8_e4m3fn),
                    out_emb.astype(jnp.float8_e4m3fn))
    return _tc_loss(dots, p, n)


# speedup vs baseline: 1.3701x; 1.0473x over previous
import functools

import jax
import jax.numpy as jnp
from jax import lax
from jax.experimental import pallas as pl
from jax.experimental.pallas import tpu as pltpu
from jax.experimental.pallas import tpu_sc as plsc

HIDDEN = 64
CTX = 128
GROUP = 4
NUM_WORKERS = 32


def _sc_dots_kernel(ex_per_w, u_labels_hbm, ctx_hbm, in_emb_hbm, out_emb_hbm,
                    out_hbm, u_idx, u_rows, lbuf, ctx_rows, dots, u_scr,
                    gsem, lsem, osem):
    wid = lax.axis_index("s") * 2 + lax.axis_index("c")
    base = wid * ex_per_w
    num_groups = ex_per_w // GROUP

    n_chunks = ex_per_w // 128
    for j in range(n_chunks):
        pltpu.sync_copy(u_labels_hbm.at[pl.ds(base + j * 128, 128)],
                        u_idx.at[j])
    cps = [pltpu.async_copy(in_emb_hbm.at[u_idx.at[j]],
                            u_rows.at[pl.ds(j * 128, 128)], gsem)
           for j in range(n_chunks)]
    for cp in cps:
        cp.wait()

    def issue_gathers(g, buf):
        for e in range(GROUP):
            pltpu.async_copy(out_emb_hbm.at[lbuf.at[buf, e]],
                             ctx_rows.at[buf, e], gsem)

    def drain_gathers(buf):
        for e in range(GROUP):
            pltpu.make_async_copy(out_emb_hbm.at[lbuf.at[buf, e]],
                                  ctx_rows.at[buf, e], gsem).wait()

    def unpack4(v64):
        b_lo, b_hi = plsc.unpack(v64, format=plsc.PackFormat.INTERLEAVED,
                                 preferred_element_type=jnp.bfloat16)
        a0, a1 = plsc.unpack(b_lo, format=plsc.PackFormat.INTERLEAVED)
        a2, a3 = plsc.unpack(b_hi, format=plsc.PackFormat.INTERLEAVED)
        return a0, a1, a2, a3

    lvec = lax.iota(jnp.int32, 16)
    m4 = lvec % 4
    sm = (m4 % 2) * 2 + m4 // 2
    uscat_idx = [sm * 16 + 4 * h + lvec // 4 for h in range(4)]

    def compute(g, buf):
        lane = lax.iota(jnp.int32, 16)
        for e in range(GROUP):
            b_local = g * GROUP + e
            for h in range(4):
                plsc.store_scatter(u_scr, [uscat_idx[h]],
                                   u_rows[b_local, pl.ds(16 * h, 16)])
            u_vecs = [u_scr[pl.ds(16 * k, 16)] for k in range(4)]

            def blk_body(t, _):
                acc = jnp.zeros((16,), jnp.float32)
                for r in range(16):
                    row = t * 16 + r
                    c_vecs = unpack4(ctx_rows[buf, e, row, pl.ds(0, 64)])
                    p = c_vecs[0] * u_vecs[0]
                    for h in range(1, 4):
                        p = p + c_vecs[h] * u_vecs[h]
                    acc = jnp.where(lane == r, jnp.sum(p), acc)
                dots[buf, e, pl.ds(t * 16, 16)] = acc
                return 0

            lax.fori_loop(0, CTX // 16, blk_body, 0)

    def issue_out(g, buf):
        pltpu.async_copy(dots.at[buf],
                         out_hbm.at[pl.ds(base + g * GROUP, GROUP)],
                         osem)

    def drain_out(g, buf):
        pltpu.make_async_copy(dots.at[buf],
                              out_hbm.at[pl.ds(base + g * GROUP, GROUP)],
                              osem).wait()

    def step(g, buf, has_next, has_prev_out):
        if has_next:
            lcp = pltpu.async_copy(ctx_hbm.at[pl.ds(base + (g + 1) * GROUP,
                                                    GROUP)],
                                   lbuf.at[1 - buf], lsem)
        drain_gathers(buf)
        if has_next:
            lcp.wait()
            issue_gathers(g + 1, 1 - buf)
        if has_prev_out:
            drain_out(g - 2, buf)
        compute(g, buf)
        issue_out(g, buf)

    pltpu.sync_copy(ctx_hbm.at[pl.ds(base, GROUP)], lbuf.at[0])
    issue_gathers(0, 0)
    step(0, 0, True, False)
    step(1, 1, True, False)

    def pair_body(i, _):
        step(2 * i, 0, True, True)
        step(2 * i + 1, 1, True, True)
        return 0

    lax.fori_loop(1, num_groups // 2 - 1, pair_body, 0)
    step(num_groups - 2, 0, True, True)
    step(num_groups - 1, 1, False, True)
    drain_out(num_groups - 2, 0)
    drain_out(num_groups - 1, 1)


def _sc_dots(u_labels, ctx_labels, in_emb, out_emb):
    b = u_labels.shape[0]
    ex_per_w = b // NUM_WORKERS
    mesh = plsc.VectorSubcoreMesh(core_axis_name="c", subcore_axis_name="s")
    f = pl.kernel(
        functools.partial(_sc_dots_kernel, ex_per_w),
        out_type=jax.ShapeDtypeStruct((b, CTX), jnp.float32),
        mesh=mesh,
        scratch_types=[
            pltpu.VMEM((ex_per_w // 128, 128), jnp.int32),
            pltpu.VMEM((ex_per_w, HIDDEN), jnp.float32),
            pltpu.VMEM((2, GROUP, CTX), jnp.int32),
            pltpu.VMEM((2, GROUP, CTX, HIDDEN), jnp.float8_e4m3fn),
            pltpu.VMEM((2, GROUP, CTX), jnp.float32),
            pltpu.VMEM((HIDDEN,), jnp.float32),
            pltpu.SemaphoreType.DMA,
            pltpu.SemaphoreType.DMA,
            pltpu.SemaphoreType.DMA,
        ],
        compiler_params=pltpu.CompilerParams(needs_layout_passes=False,
                                             use_tc_tiling_on_sc=False),
    )
    return f(u_labels, ctx_labels, in_emb, out_emb)


def _tc_loss_kernel(p, n, d_ref, o_ref):
    d = d_ref[...]
    col = lax.broadcasted_iota(jnp.int32, d.shape, 1)
    x = jnp.where(col < p, d, -d)
    ls = jnp.minimum(x, 0.0) - jnp.log1p(jnp.exp(-jnp.abs(x)))
    ls = jnp.where(col < p + n, ls, 0.0)
    loss = -jnp.sum(ls, axis=1)
    o_ref[...] = loss.reshape(o_ref.shape)


def _tc_loss(dots, p, n):
    b = dots.shape[0]
    blk = 2048
    out = pl.pallas_call(
        functools.partial(_tc_loss_kernel, p, n),
        grid=(b // blk,),
        in_specs=[pl.BlockSpec((blk, CTX), lambda i: (i, 0))],
        out_specs=pl.BlockSpec((blk // 128, 128), lambda i: (i, 0)),
        out_shape=jax.ShapeDtypeStruct((b // 128, 128), jnp.float32),
    )(dots)
    return out.reshape(b)


def kernel(input_labels, pos_labels, neg_labels, in_emb, out_emb):
    b, p = pos_labels.shape
    n = neg_labels.shape[1]
    pad = CTX - p - n
    ctx = jnp.concatenate(
        [pos_labels.astype(jnp.int32), neg_labels.astype(jnp.int32),
         jnp.zeros((b, pad), jnp.int32)], axis=1)
    dots = _sc_dots(input_labels.astype(jnp.int32), ctx,
                    in_emb, out_emb.astype(jnp.float8_e4m3fn))
    return _tc_loss(dots, p, n)


# speedup vs baseline: 1.7094x; 1.2476x over previous
import functools

import jax
import jax.numpy as jnp
from jax import lax
from jax.experimental import pallas as pl
from jax.experimental.pallas import tpu as pltpu
from jax.experimental.pallas import tpu_sc as plsc

HIDDEN = 64
CTX = 128
CGW = 120
GROUP = 4
NUM_WORKERS = 32


def _sc_dots_kernel(ex_per_w, u_labels_hbm, ctx_hbm, in_emb_hbm, out_emb_hbm,
                    out_hbm, u_idx, u_rows, lbuf, ctx_rows, dots, u_scr,
                    gsem, lsem, osem):
    wid = lax.axis_index("s") * 2 + lax.axis_index("c")
    base = wid * ex_per_w
    num_groups = ex_per_w // GROUP

    n_chunks = ex_per_w // 128
    for j in range(n_chunks):
        pltpu.sync_copy(u_labels_hbm.at[pl.ds(base + j * 128, 128)],
                        u_idx.at[j])
    cps = [pltpu.async_copy(in_emb_hbm.at[u_idx.at[j]],
                            u_rows.at[pl.ds(j * 128, 128)], gsem)
           for j in range(n_chunks)]
    for cp in cps:
        cp.wait()

    def issue_gathers(g, buf):
        for e in range(GROUP):
            pltpu.async_copy(out_emb_hbm.at[lbuf.at[buf, e]],
                             ctx_rows.at[buf, e, pl.ds(0, CGW)], gsem)

    def drain_gathers(buf):
        for e in range(GROUP):
            pltpu.make_async_copy(out_emb_hbm.at[lbuf.at[buf, e]],
                                  ctx_rows.at[buf, e, pl.ds(0, CGW)],
                                  gsem).wait()

    def unpack4(v64):
        b_lo, b_hi = plsc.unpack(v64, format=plsc.PackFormat.INTERLEAVED,
                                 preferred_element_type=jnp.bfloat16)
        a0, a1 = plsc.unpack(b_lo, format=plsc.PackFormat.INTERLEAVED)
        a2, a3 = plsc.unpack(b_hi, format=plsc.PackFormat.INTERLEAVED)
        return a0, a1, a2, a3

    lvec = lax.iota(jnp.int32, 16)
    m4 = lvec % 4
    sm = (m4 % 2) * 2 + m4 // 2
    uscat_idx = [sm * 16 + 4 * h + lvec // 4 for h in range(4)]

    def compute(g, buf):
        lane = lax.iota(jnp.int32, 16)
        for e in range(GROUP):
            b_local = g * GROUP + e
            for h in range(4):
                plsc.store_scatter(u_scr, [uscat_idx[h]],
                                   u_rows[b_local, pl.ds(16 * h, 16)])
            u_vecs = [u_scr[pl.ds(16 * k, 16)] for k in range(4)]

            def blk_body(t, _):
                acc = jnp.zeros((16,), jnp.float32)
                for r in range(16):
                    row = t * 16 + r
                    c_vecs = unpack4(ctx_rows[buf, e, row, pl.ds(0, 64)])
                    p = c_vecs[0] * u_vecs[0]
                    for h in range(1, 4):
                        p = p + c_vecs[h] * u_vecs[h]
                    acc = jnp.where(lane == r, jnp.sum(p), acc)
                dots[buf, e, pl.ds(t * 16, 16)] = acc
                return 0

            lax.fori_loop(0, CTX // 16, blk_body, 0)

    def issue_out(g, buf):
        pltpu.async_copy(dots.at[buf],
                         out_hbm.at[pl.ds(base + g * GROUP, GROUP)],
                         osem)

    def drain_out(g, buf):
        pltpu.make_async_copy(dots.at[buf],
                              out_hbm.at[pl.ds(base + g * GROUP, GROUP)],
                              osem).wait()

    def step(g, buf, has_next, has_prev_out):
        if has_next:
            lcp = pltpu.async_copy(ctx_hbm.at[pl.ds(base + (g + 1) * GROUP,
                                                    GROUP)],
                                   lbuf.at[1 - buf], lsem)
        drain_gathers(buf)
        if has_next:
            lcp.wait()
            issue_gathers(g + 1, 1 - buf)
        if has_prev_out:
            drain_out(g - 2, buf)
        compute(g, buf)
        issue_out(g, buf)

    pltpu.sync_copy(ctx_hbm.at[pl.ds(base, GROUP)], lbuf.at[0])
    issue_gathers(0, 0)
    step(0, 0, True, False)
    step(1, 1, True, False)

    def pair_body(i, _):
        step(2 * i, 0, True, True)
        step(2 * i + 1, 1, True, True)
        return 0

    lax.fori_loop(1, num_groups // 2 - 1, pair_body, 0)
    step(num_groups - 2, 0, True, True)
    step(num_groups - 1, 1, False, True)
    drain_out(num_groups - 2, 0)
    drain_out(num_groups - 1, 1)


def _sc_dots(u_labels, ctx_labels, in_emb, out_emb):
    b = u_labels.shape[0]
    ex_per_w = b // NUM_WORKERS
    mesh = plsc.VectorSubcoreMesh(core_axis_name="c", subcore_axis_name="s")
    f = pl.kernel(
        functools.partial(_sc_dots_kernel, ex_per_w),
        out_type=jax.ShapeDtypeStruct((b, CTX), jnp.float32),
        mesh=mesh,
        scratch_types=[
            pltpu.VMEM((ex_per_w // 128, 128), jnp.int32),
            pltpu.VMEM((ex_per_w, HIDDEN), jnp.float32),
            pltpu.VMEM((2, GROUP, CGW), jnp.int32),
            pltpu.VMEM((2, GROUP, CTX, HIDDEN), jnp.float8_e4m3fn),
            pltpu.VMEM((2, GROUP, CTX), jnp.float32),
            pltpu.VMEM((HIDDEN,), jnp.float32),
            pltpu.SemaphoreType.DMA,
            pltpu.SemaphoreType.DMA,
            pltpu.SemaphoreType.DMA,
        ],
        compiler_params=pltpu.CompilerParams(needs_layout_passes=False,
                                             use_tc_tiling_on_sc=False),
    )
    return f(u_labels, ctx_labels, in_emb, out_emb)


def _tc_loss_kernel(p, n, d_ref, o_ref):
    d = d_ref[...]
    col = lax.broadcasted_iota(jnp.int32, d.shape, 1)
    x = jnp.where(col < p, d, -d)
    ls = jnp.minimum(x, 0.0) - jnp.log1p(jnp.exp(-jnp.abs(x)))
    ls = jnp.where(col < p + n, ls, 0.0)
    loss = -jnp.sum(ls, axis=1)
    o_ref[...] = loss.reshape(o_ref.shape)


def _tc_loss(dots, p, n):
    b = dots.shape[0]
    blk = 2048
    out = pl.pallas_call(
        functools.partial(_tc_loss_kernel, p, n),
        grid=(b // blk,),
        in_specs=[pl.BlockSpec((blk, CTX), lambda i: (i, 0))],
        out_specs=pl.BlockSpec((blk // 128, 128), lambda i: (i, 0)),
        out_shape=jax.ShapeDtypeStruct((b // 128, 128), jnp.float32),
    )(dots)
    return out.reshape(b)


def kernel(input_labels, pos_labels, neg_labels, in_emb, out_emb):
    b, p = pos_labels.shape
    n = neg_labels.shape[1]
    ctx = jnp.concatenate(
        [pos_labels.astype(jnp.int32), neg_labels.astype(jnp.int32)], axis=1)
    dots = _sc_dots(input_labels.astype(jnp.int32), ctx,
                    in_emb, out_emb.astype(jnp.float8_e4m3fn))
    return _tc_loss(dots, p, n)
